# baseline (speedup 1.0000x reference)
import functools

import jax
import jax.numpy as jnp
from jax import lax
from jax.experimental import pallas as pl
from jax.experimental.pallas import tpu as pltpu
from jax.experimental.pallas import tpu_sc as plsc

DIM = 64
NC = 2
NS = 16
NW = NC * NS
HB = 8
BW = 512
HALF = 256


def _sc_gather(xt3, table128, R, H):
    n_hb = H // HB
    mesh = plsc.VectorSubcoreMesh(core_axis_name="c", subcore_axis_name="s")

    @functools.partial(
        pl.kernel,
        mesh=mesh,
        compiler_params=pltpu.CompilerParams(needs_layout_passes=False),
        out_type=jax.ShapeDtypeStruct((H, DIM, R), jnp.float32),
        scratch_types=[
            pltpu.VMEM((HB, 8, 128), jnp.int32),
            pltpu.VMEM((2, HALF, 128), jnp.float32),
            pltpu.VMEM((2, DIM, HALF), jnp.float32),
            pltpu.SemaphoreType.DMA,
            pltpu.SemaphoreType.DMA((2,)),
            pltpu.SemaphoreType.DMA((2,)),
        ],
    )
    def k(table_hbm, xt_hbm, out_hbm, idx_v, rows_v, tout_v,
          sem_x, sem_g, sem_w):
        wid = lax.axis_index("s") * NC + lax.axis_index("c")
        b0 = wid * BW
        bhalf = 4 * lax.rem(wid, 2)
        d1off = lax.div(wid, 2) * 8
        li = lax.iota(jnp.int32, 16)
        iotas = [li + 16 * g for g in range(16)]


        def stage_idx(hb):
            pltpu.sync_copy(
                xt_hbm.at[pl.ds(hb * HB, HB), pl.ds(d1off, 8), :], idx_v)

        def start_gathers(q, j, ub):
            for i in range(2):
                pltpu.async_copy(
                    table_hbm.at[idx_v.at[q, bhalf + 2 * j + i]],
                    rows_v.at[ub, pl.ds(128 * i, 128)], sem_g.at[ub])

        def wait_gathers(ub):
            for i in range(2):
                pltpu.make_async_copy(
                    table_hbm.at[pl.ds(0, 128)],
                    rows_v.at[ub, pl.ds(128 * i, 128)],
                    sem_g.at[ub]).wait()

        def start_write(h_abs, j, ub):
            pltpu.async_copy(tout_v.at[ub],
                             out_hbm.at[h_abs, :, pl.ds(b0 + HALF * j, HALF)],
                             sem_w.at[ub])

        def wait_write(ub):
            pltpu.make_async_copy(out_hbm.at[0, :, pl.ds(0, HALF)],
                                  tout_v.at[ub], sem_w.at[ub]).wait()

        def transpose(ub):
            def bloop(b, c):
                bsplat = jnp.full((16,), b, jnp.int32)
                for j in range(DIM // 16):
                    plsc.store_scatter(
                        tout_v.at[ub], [iotas[j], bsplat],
                        rows_v[ub, b, pl.ds(16 * j, 16)])
                return c

            lax.fori_loop(0, HALF, bloop, 0)

        stage_idx(0)
        start_gathers(0, 0, 0)
        start_gathers(0, 1, 1)

        def block(hb, c):
            def upair(p, cc):
                for j in range(2):
                    ub = j
                    gu = hb * 16 + 2 * p + j
                    wait_gathers(ub)

                    @pl.when(gu >= 2)
                    def _():
                        wait_write(ub)

                    transpose(ub)
                    start_write(hb * HB + p, j, ub)

                    @pl.when(p + 1 < HB)
                    def _():
                        start_gathers(p + 1, j, ub)

                @pl.when((p + 1 >= HB) & (hb + 1 < n_hb))
                def _():
                    stage_idx(hb + 1)
                    start_gathers(0, 0, 0)
                    start_gathers(0, 1, 1)

                return cc

            lax.fori_loop(0, HB, upair, 0)
            return c

        lax.fori_loop(0, n_hb, block, 0)
        wait_write(0)
        wait_write(1)

    return k(table128, xt3)


def kernel(x, embedding):
    R, H = x.shape
    table128 = jnp.pad(embedding, ((0, 0), (0, 128 - DIM)))
    xt3 = x.T.reshape(H, R // 128, 128)
    out_t = _sc_gather(xt3, table128, R, H)
    return jnp.transpose(out_t, (2, 0, 1))


# speedup vs baseline: 1.9166x; 1.9166x over previous
import functools

import jax
import jax.numpy as jnp
from jax import lax
from jax.experimental import pallas as pl
from jax.experimental.pallas import tpu as pltpu
from jax.experimental.pallas import tpu_sc as plsc

DIM = 64
NC = 2
NS = 16
NW = NC * NS
CW = 128
SUP = 32


def _sc_gather(x, table128):
    R, H = x.shape
    B = R * H
    rows_per_w = R // NW
    n_sup = rows_per_w // SUP
    n_ch = SUP * H // CW
    assert n_sup % 2 == 0 and n_ch % 2 == 0 and (SUP * H) % CW == 0
    mesh = plsc.VectorSubcoreMesh(core_axis_name="c", subcore_axis_name="s")

    @functools.partial(
        pl.kernel,
        mesh=mesh,
        out_type=jax.ShapeDtypeStruct((B, DIM), jnp.float32),
        scratch_types=[
            pltpu.VMEM((2, SUP, H), jnp.int32),
            pltpu.VMEM((2, n_ch, CW), jnp.int32),
            pltpu.VMEM((2, CW, 128), jnp.float32),
            pltpu.VMEM((2, CW, DIM), jnp.float32),
            pltpu.SemaphoreType.DMA((2,)),
            pltpu.SemaphoreType.DMA((2,)),
            pltpu.SemaphoreType.DMA((2,)),
        ],
    )
    def k(table_hbm, x_hbm, out_hbm, raw_v, flat_v, rows_v, out_v,
          sem_x, sem_g, sem_w):
        wid = lax.axis_index("s") * NC + lax.axis_index("c")
        xrow0 = wid * rows_per_w
        obase = xrow0 * H

        li = lax.iota(jnp.int32, 16)
        mlo = li < 8
        mhi = li >= 8

        def start_x(s, rb):
            pltpu.async_copy(x_hbm.at[pl.ds(xrow0 + s * SUP, SUP)],
                             raw_v.at[rb], sem_x.at[rb])

        def wait_x(rb):
            pltpu.make_async_copy(x_hbm.at[pl.ds(0, SUP)],
                                  raw_v.at[rb], sem_x.at[rb]).wait()

        def compact_idx(rb, fb):
            for m in range(SUP * H // 16):
                q = 16 * m
                r, o = q // H, q % H
                cc, lane = q // CW, q % CW
                if o <= H - 16:
                    flat_v[fb, cc, pl.ds(lane, 16)] = \
                        raw_v[rb, r, pl.ds(o, 16)]
                else:
                    a = raw_v[rb, r, pl.ds(H - 16, 16)]
                    bv = raw_v[rb, r + 1, pl.ds(0, 16)]
                    hi = a.at[jnp.minimum(li + 8, 15)].get(
                        mode="promise_in_bounds")
                    lo = bv.at[jnp.maximum(li - 8, 0)].get(
                        mode="promise_in_bounds")
                    flat_v[fb, cc, pl.ds(lane, 16)] = jnp.where(mlo, hi, lo)

        def start_gather(fb, c, b):
            pltpu.async_copy(table_hbm.at[flat_v.at[fb, c]],
                             rows_v.at[b], sem_g.at[b])

        def wait_gather(b):
            pltpu.make_async_copy(table_hbm.at[pl.ds(0, CW)],
                                  rows_v.at[b], sem_g.at[b]).wait()

        def start_write(s, c, b):
            pltpu.async_copy(out_v.at[b],
                             out_hbm.at[pl.ds(obase + (s * n_ch + c) * CW,
                                              CW)],
                             sem_w.at[b])

        def wait_write(b):
            pltpu.make_async_copy(out_hbm.at[pl.ds(0, CW)],
                                  out_v.at[b], sem_w.at[b]).wait()

        RU = 8

        def compact_rows(b):
            def rowblk(i, cc):
                r0 = i * RU
                for kk in range(RU):
                    for j in range(DIM // 16):
                        out_v[b, r0 + kk, pl.ds(j * 16, 16)] = \
                            rows_v[b, r0 + kk, pl.ds(j * 16, 16)]
                return cc

            lax.fori_loop(0, CW // RU, rowblk, 0)

        def sup_body(s, sb):
            @pl.when(s + 1 < n_sup)
            def _():
                wait_x(1 - sb)

            if True:
                @pl.when(s + 1 < n_sup)
                def _():
                    compact_idx(1 - sb, 1 - sb)

                @pl.when(s + 2 < n_sup)
                def _():
                    start_x(s + 2, sb)

            def step(c, b):
                wait_gather(b)

                @pl.when(c + 1 < n_ch)
                def _():
                    start_gather(sb, c + 1, 1 - b)

                @pl.when((c + 1 >= n_ch) & (s + 1 < n_sup))
                def _():
                    start_gather(1 - sb, 0, 1 - b)

                @pl.when((s > 0) | (c >= 2))
                def _():
                    wait_write(b)

                compact_rows(b)
                start_write(s, c, b)

            def chpair(p, cc):
                step(2 * p, 0)
                step(2 * p + 1, 1)
                return cc

            lax.fori_loop(0, n_ch // 2, chpair, 0)

        start_x(0, 0)
        wait_x(0)
        start_x(1, 1)
        compact_idx(0, 0)
        start_gather(0, 0, 0)

        def sup_pair(sp, cc):
            sup_body(2 * sp, 0)
            sup_body(2 * sp + 1, 1)
            return cc

        lax.fori_loop(0, n_sup // 2, sup_pair, 0)
        wait_write(0)
        wait_write(1)

    return k(table128, x)


def kernel(x, embedding):
    b, h = x.shape
    table128 = jnp.pad(embedding, ((0, 0), (0, 128 - DIM)))
    out = _sc_gather(x, table128)
    return out.reshape(b, h, DIM)
